# baseline (speedup 1.0000x reference)
import functools

import jax
import jax.numpy as jnp
from jax import lax
from jax.experimental import pallas as pl
from jax.experimental.pallas import tpu as pltpu
from jax.experimental.pallas import tpu_sc as plsc

N_NODES = 10000
N_EDGES = 320000
F_IN = 128
F_HID = 256
F_OUT = 128

NPAD = 10240
PAD_NODE = N_NODES
NC = 2
NS = 16
NW = NC * NS
CHUNK = 128
CHUNKS = 80
EPAD = NW * CHUNKS * CHUNK
RPT = NPAD // NS

_mesh = plsc.VectorSubcoreMesh(core_axis_name="c", subcore_axis_name="s")


@functools.partial(
    pl.kernel,
    out_type=jax.ShapeDtypeStruct((NC, 2, NPAD), jnp.float32),
    mesh=_mesh,
    scratch_types=[
        pltpu.VMEM((CHUNKS, CHUNK), jnp.int32),
        pltpu.VMEM((CHUNKS, CHUNK), jnp.int32),
        pltpu.VMEM((CHUNK,), jnp.float32),
        pltpu.VMEM_SHARED((NPAD,), jnp.float32),
        pltpu.VMEM_SHARED((NPAD,), jnp.float32),
        pltpu.SemaphoreType.DMA,
        pltpu.SemaphoreType.DMA,
    ],
)
def _deg_kernel(src_hbm, dst_hbm, zeros_hbm, out_hbm,
                src_v, dst_v, ones_v, acc_out, acc_in, dsem0, dsem1):
    cid = lax.axis_index("c")
    sid = lax.axis_index("s")
    wid = sid * NC + cid
    pltpu.sync_copy(src_hbm.at[wid], src_v)
    pltpu.sync_copy(dst_hbm.at[wid], dst_v)
    for i in range(CHUNK // 16):
        ones_v[pl.ds(16 * i, 16)] = jnp.ones((16,), jnp.float32)
    pltpu.sync_copy(zeros_hbm.at[pl.ds(sid * RPT, RPT)],
                    acc_out.at[pl.ds(sid * RPT, RPT)])
    pltpu.sync_copy(zeros_hbm.at[pl.ds(sid * RPT, RPT)],
                    acc_in.at[pl.ds(sid * RPT, RPT)])
    plsc.subcore_barrier()

    @pl.loop(0, CHUNKS)
    def _(c):
        pltpu.async_copy(ones_v, acc_out.at[src_v.at[c]], dsem0, add=True)
        pltpu.async_copy(ones_v, acc_in.at[dst_v.at[c]], dsem1, add=True)

    @pl.loop(0, CHUNKS)
    def _(c):
        pltpu.make_async_copy(ones_v, acc_out.at[src_v.at[c]], dsem0).wait()
        pltpu.make_async_copy(ones_v, acc_in.at[dst_v.at[c]], dsem1).wait()

    plsc.subcore_barrier()
    pltpu.sync_copy(acc_out.at[pl.ds(sid * RPT, RPT)],
                    out_hbm.at[cid, 0, pl.ds(sid * RPT, RPT)])
    pltpu.sync_copy(acc_in.at[pl.ds(sid * RPT, RPT)],
                    out_hbm.at[cid, 1, pl.ds(sid * RPT, RPT)])


FH = 64


def _agg_phase_body(h, cid, sid, table_hbm, zeros_hbm, out_hbm,
                    src_v, dst_v, rows_v, acc, tab_s, sems, a_sm):
    pltpu.sync_copy(zeros_hbm.at[pl.ds(sid * RPT, RPT)],
                    acc.at[pl.ds(sid * RPT, RPT)])
    if a_sm is None:
        pltpu.sync_copy(table_hbm.at[h, pl.ds(sid * RPT, RPT)],
                        tab_s.at[pl.ds(sid * RPT, RPT)])
    else:
        for k in range(RPT // CHUNK):
            r0 = sid * RPT + k * CHUNK
            pltpu.sync_copy(table_hbm.at[h, pl.ds(r0, CHUNK)], rows_v.at[0])

            @pl.loop(0, CHUNK // 16)
            def _(g):
                av = a_sm[pl.ds(k * CHUNK + g * 16, 16)]
                for i in range(16):
                    sv = jnp.full((16,), av[i], jnp.float32)
                    r = g * 16 + i
                    for j in range(FH // 16):
                        rows_v[0, r, pl.ds(16 * j, 16)] = (
                            rows_v[0, r, pl.ds(16 * j, 16)] * sv)

            pltpu.sync_copy(rows_v.at[0], tab_s.at[pl.ds(r0, CHUNK)])
    plsc.subcore_barrier()

    pltpu.async_copy(tab_s.at[src_v.at[0]], rows_v.at[0], sems[0])

    @pl.loop(0, CHUNKS, step=2)
    def _(c):
        for b in range(2):
            cc = c + b
            nxt = cc + 1

            @pl.when(nxt < CHUNKS)
            def _():
                pltpu.async_copy(tab_s.at[src_v.at[nxt]],
                                 rows_v.at[1 - b], sems[1 - b])

            pltpu.make_async_copy(tab_s.at[src_v.at[cc]],
                                  rows_v.at[b], sems[b]).wait()
            pltpu.sync_copy(rows_v.at[b], acc.at[dst_v.at[cc]], add=True)

    plsc.subcore_barrier()
    pltpu.sync_copy(acc.at[pl.ds(sid * RPT, RPT)],
                    out_hbm.at[cid, h, pl.ds(sid * RPT, RPT)])
    plsc.subcore_barrier()


_AGG_SCRATCH = [
    pltpu.VMEM((CHUNKS, CHUNK), jnp.int32),
    pltpu.VMEM((CHUNKS, CHUNK), jnp.int32),
    pltpu.VMEM((2, CHUNK, FH), jnp.float32),
    pltpu.VMEM_SHARED((NPAD, FH), jnp.float32),
    pltpu.VMEM_SHARED((NPAD, FH), jnp.float32),
    pltpu.SemaphoreType.DMA,
    pltpu.SemaphoreType.DMA,
]


@functools.partial(
    pl.kernel,
    out_type=jax.ShapeDtypeStruct((NC, 2, NPAD, FH), jnp.float32),
    mesh=_mesh,
    scratch_types=_AGG_SCRATCH + [pltpu.VMEM((RPT,), jnp.float32)],
    compiler_params=pltpu.CompilerParams(use_tc_tiling_on_sc=False),
)
def _agg1_kernel(table_hbm, src_hbm, dst_hbm, zeros_hbm, a_hbm, out_hbm,
                 src_v, dst_v, rows_v, acc, tab_s, sem0, sem1, a_sm):
    cid = lax.axis_index("c")
    sid = lax.axis_index("s")
    wid = sid * NC + cid
    pltpu.sync_copy(src_hbm.at[wid], src_v)
    pltpu.sync_copy(dst_hbm.at[wid], dst_v)
    pltpu.sync_copy(a_hbm.at[pl.ds(sid * RPT, RPT)], a_sm)
    for h in range(2):
        _agg_phase_body(h, cid, sid, table_hbm, zeros_hbm, out_hbm,
                        src_v, dst_v, rows_v, acc, tab_s, (sem0, sem1), a_sm)


@functools.partial(
    pl.kernel,
    out_type=jax.ShapeDtypeStruct((NC, 2, NPAD, FH), jnp.float32),
    mesh=_mesh,
    scratch_types=list(_AGG_SCRATCH),
    compiler_params=pltpu.CompilerParams(use_tc_tiling_on_sc=False),
)
def _agg_kernel(table_hbm, src_hbm, dst_hbm, zeros_hbm, out_hbm,
                src_v, dst_v, rows_v, acc, tab_s, sem0, sem1):
    cid = lax.axis_index("c")
    sid = lax.axis_index("s")
    wid = sid * NC + cid
    pltpu.sync_copy(src_hbm.at[wid], src_v)
    pltpu.sync_copy(dst_hbm.at[wid], dst_v)
    for h in range(2):
        _agg_phase_body(h, cid, sid, table_hbm, zeros_hbm, out_hbm,
                        src_v, dst_v, rows_v, acc, tab_s, (sem0, sem1), None)


def _prep_body(dp_ref, a_ref, b_ref):
    dp = dp_ref[...]
    out_deg = jnp.transpose(dp[0:1] + dp[2:3])
    in_deg = jnp.transpose(dp[1:2] + dp[3:4])
    a_ref[...] = lax.rsqrt(jnp.maximum(out_deg, 1.0))
    b_ref[...] = lax.rsqrt(jnp.maximum(in_deg, 1.0))


_prep_call = pl.pallas_call(
    _prep_body,
    out_shape=[
        jax.ShapeDtypeStruct((NPAD, 1), jnp.float32),
        jax.ShapeDtypeStruct((NPAD, 1), jnp.float32),
    ],
)

BM = 1024


def _mid_body(p, a, b, W1, b1, W2, m2):
    agg = jnp.concatenate([p[0, 0] + p[1, 0], p[0, 1] + p[1, 1]], axis=1)
    agg = agg * b[...]
    h = jnp.dot(agg, W1[...], preferred_element_type=jnp.float32) + b1[...]
    h = jnp.where(h >= 0, h, 0.01 * h)
    g = jnp.dot(h, W2[...], preferred_element_type=jnp.float32)
    g = g * a[...]
    m2[0] = g[:, :FH]
    m2[1] = g[:, FH:]


_mid_call = pl.pallas_call(
    _mid_body,
    grid=(NPAD // BM,),
    in_specs=[
        pl.BlockSpec((NC, 2, BM, FH), lambda i: (0, 0, i, 0)),
        pl.BlockSpec((BM, 1), lambda i: (i, 0)),
        pl.BlockSpec((BM, 1), lambda i: (i, 0)),
        pl.BlockSpec((F_IN, F_HID), lambda i: (0, 0)),
        pl.BlockSpec((1, F_HID), lambda i: (0, 0)),
        pl.BlockSpec((F_HID, F_OUT), lambda i: (0, 0)),
    ],
    out_specs=pl.BlockSpec((2, BM, FH), lambda i: (0, i, 0)),
    out_shape=jax.ShapeDtypeStruct((2, NPAD, FH), jnp.float32),
)


def _fin_body(q, b, b2, out):
    z = jnp.concatenate([q[0, 0, :N_NODES] + q[1, 0, :N_NODES],
                         q[0, 1, :N_NODES] + q[1, 1, :N_NODES]], axis=1)
    z = z * b[:N_NODES] + b2[...]
    out[...] = jax.nn.sigmoid(z)


_fin_call = pl.pallas_call(
    _fin_body,
    out_shape=jax.ShapeDtypeStruct((N_NODES, F_OUT), jnp.float32),
)


def kernel(x, edge_index, W1, b1, W2, b2):
    src = edge_index[0].astype(jnp.int32)
    dst = edge_index[1].astype(jnp.int32)
    pad = jnp.full((EPAD - N_EDGES,), PAD_NODE, jnp.int32)
    src3 = jnp.concatenate([src, pad]).reshape(NW, CHUNKS, CHUNK)
    dst3 = jnp.concatenate([dst, pad]).reshape(NW, CHUNKS, CHUNK)
    zrow = jnp.zeros((NPAD, FH), jnp.float32)
    zvec = jnp.zeros((NPAD,), jnp.float32)

    x2 = jnp.pad(x, ((0, NPAD - N_NODES), (0, 0)))
    x2 = x2.reshape(NPAD, 2, FH).transpose(1, 0, 2)

    degp = _deg_kernel(src3, dst3, zvec)
    a2, b2d = _prep_call(degp.reshape(2 * NC, NPAD))
    p = _agg1_kernel(x2, src3, dst3, zrow, a2.reshape(NPAD))
    m2 = _mid_call(p, a2, b2d, W1, b1.reshape(1, F_HID), W2)
    q = _agg_kernel(m2, src3, dst3, zrow)
    return _fin_call(q, b2d, b2.reshape(1, F_OUT))


# speedup vs baseline: 1.0114x; 1.0114x over previous
import functools

import jax
import jax.numpy as jnp
from jax import lax
from jax.experimental import pallas as pl
from jax.experimental.pallas import tpu as pltpu
from jax.experimental.pallas import tpu_sc as plsc

N_NODES = 10000
N_EDGES = 320000
F_IN = 128
F_HID = 256
F_OUT = 128

NPAD = 10240
PAD_NODE = N_NODES
NC = 2
NS = 16
NW = NC * NS
CHUNK = 128
CHUNKS = 80
EPAD = NW * CHUNKS * CHUNK
RPT = NPAD // NS

_mesh = plsc.VectorSubcoreMesh(core_axis_name="c", subcore_axis_name="s")


@functools.partial(
    pl.kernel,
    out_type=jax.ShapeDtypeStruct((NC, 2, NPAD), jnp.float32),
    mesh=_mesh,
    scratch_types=[
        pltpu.VMEM((CHUNKS, CHUNK), jnp.int32),
        pltpu.VMEM((CHUNKS, CHUNK), jnp.int32),
        pltpu.VMEM((CHUNK,), jnp.float32),
        pltpu.VMEM_SHARED((NPAD,), jnp.float32),
        pltpu.VMEM_SHARED((NPAD,), jnp.float32),
        pltpu.SemaphoreType.DMA,
        pltpu.SemaphoreType.DMA,
    ],
)
def _deg_kernel(src_hbm, dst_hbm, zeros_hbm, out_hbm,
                src_v, dst_v, ones_v, acc_out, acc_in, dsem0, dsem1):
    cid = lax.axis_index("c")
    sid = lax.axis_index("s")
    wid = sid * NC + cid
    pltpu.sync_copy(src_hbm.at[wid], src_v)
    pltpu.sync_copy(dst_hbm.at[wid], dst_v)
    for i in range(CHUNK // 16):
        ones_v[pl.ds(16 * i, 16)] = jnp.ones((16,), jnp.float32)
    pltpu.sync_copy(zeros_hbm.at[pl.ds(sid * RPT, RPT)],
                    acc_out.at[pl.ds(sid * RPT, RPT)])
    pltpu.sync_copy(zeros_hbm.at[pl.ds(sid * RPT, RPT)],
                    acc_in.at[pl.ds(sid * RPT, RPT)])
    plsc.subcore_barrier()

    @pl.loop(0, CHUNKS)
    def _(c):
        pltpu.async_copy(ones_v, acc_out.at[src_v.at[c]], dsem0, add=True)
        pltpu.async_copy(ones_v, acc_in.at[dst_v.at[c]], dsem1, add=True)

    @pl.loop(0, CHUNKS)
    def _(c):
        pltpu.make_async_copy(ones_v, acc_out.at[src_v.at[c]], dsem0).wait()
        pltpu.make_async_copy(ones_v, acc_in.at[dst_v.at[c]], dsem1).wait()

    plsc.subcore_barrier()
    pltpu.sync_copy(acc_out.at[pl.ds(sid * RPT, RPT)],
                    out_hbm.at[cid, 0, pl.ds(sid * RPT, RPT)])
    pltpu.sync_copy(acc_in.at[pl.ds(sid * RPT, RPT)],
                    out_hbm.at[cid, 1, pl.ds(sid * RPT, RPT)])


FH = 64


def _agg_phase_body(h, cid, sid, table_hbm, zeros_hbm, out_hbm,
                    src_v, dst_v, rows_v, acc, tab_s, sems):
    pltpu.sync_copy(zeros_hbm.at[pl.ds(sid * RPT, RPT)],
                    acc.at[pl.ds(sid * RPT, RPT)])
    pltpu.sync_copy(table_hbm.at[h, pl.ds(sid * RPT, RPT)],
                    tab_s.at[pl.ds(sid * RPT, RPT)])
    plsc.subcore_barrier()

    pltpu.async_copy(tab_s.at[src_v.at[0]], rows_v.at[0], sems[0])

    @pl.loop(0, CHUNKS, step=2)
    def _(c):
        for b in range(2):
            cc = c + b
            nxt = cc + 1

            @pl.when(nxt < CHUNKS)
            def _():
                pltpu.async_copy(tab_s.at[src_v.at[nxt]],
                                 rows_v.at[1 - b], sems[1 - b])

            pltpu.make_async_copy(tab_s.at[src_v.at[cc]],
                                  rows_v.at[b], sems[b]).wait()
            pltpu.sync_copy(rows_v.at[b], acc.at[dst_v.at[cc]], add=True)

    plsc.subcore_barrier()
    pltpu.sync_copy(acc.at[pl.ds(sid * RPT, RPT)],
                    out_hbm.at[cid, h, pl.ds(sid * RPT, RPT)])
    plsc.subcore_barrier()


_AGG_SCRATCH = [
    pltpu.VMEM((CHUNKS, CHUNK), jnp.int32),
    pltpu.VMEM((CHUNKS, CHUNK), jnp.int32),
    pltpu.VMEM((2, CHUNK, FH), jnp.float32),
    pltpu.VMEM_SHARED((NPAD, FH), jnp.float32),
    pltpu.VMEM_SHARED((NPAD, FH), jnp.float32),
    pltpu.SemaphoreType.DMA,
    pltpu.SemaphoreType.DMA,
]


@functools.partial(
    pl.kernel,
    out_type=jax.ShapeDtypeStruct((NC, 2, NPAD, FH), jnp.float32),
    mesh=_mesh,
    scratch_types=list(_AGG_SCRATCH),
    compiler_params=pltpu.CompilerParams(use_tc_tiling_on_sc=False),
)
def _agg_kernel(table_hbm, src_hbm, dst_hbm, zeros_hbm, out_hbm,
                src_v, dst_v, rows_v, acc, tab_s, sem0, sem1):
    cid = lax.axis_index("c")
    sid = lax.axis_index("s")
    wid = sid * NC + cid
    pltpu.sync_copy(src_hbm.at[wid], src_v)
    pltpu.sync_copy(dst_hbm.at[wid], dst_v)
    for h in range(2):
        _agg_phase_body(h, cid, sid, table_hbm, zeros_hbm, out_hbm,
                        src_v, dst_v, rows_v, acc, tab_s, (sem0, sem1))


def _prep_body(dp_ref, x_ref, m1_ref, a_ref, b_ref):
    dp = dp_ref[...]
    out_deg = jnp.transpose(dp[0:1] + dp[2:3])
    in_deg = jnp.transpose(dp[1:2] + dp[3:4])
    a = lax.rsqrt(jnp.maximum(out_deg, 1.0))
    b = lax.rsqrt(jnp.maximum(in_deg, 1.0))
    a_ref[...] = a
    b_ref[...] = b
    m1 = x_ref[...] * a[:N_NODES]
    m1_ref[0, :N_NODES] = m1[:, :FH]
    m1_ref[1, :N_NODES] = m1[:, FH:]
    m1_ref[0, N_NODES:] = jnp.zeros((NPAD - N_NODES, FH), jnp.float32)
    m1_ref[1, N_NODES:] = jnp.zeros((NPAD - N_NODES, FH), jnp.float32)


_prep_call = pl.pallas_call(
    _prep_body,
    out_shape=[
        jax.ShapeDtypeStruct((2, NPAD, FH), jnp.float32),
        jax.ShapeDtypeStruct((NPAD, 1), jnp.float32),
        jax.ShapeDtypeStruct((NPAD, 1), jnp.float32),
    ],
)

BM = 1024


def _mid_body(p, a, b, W1, b1, W2, m2):
    agg = jnp.concatenate([p[0, 0] + p[1, 0], p[0, 1] + p[1, 1]], axis=1)
    agg = agg * b[...]
    h = jnp.dot(agg, W1[...], preferred_element_type=jnp.float32) + b1[...]
    h = jnp.where(h >= 0, h, 0.01 * h)
    g = jnp.dot(h, W2[...], preferred_element_type=jnp.float32)
    g = g * a[...]
    m2[0] = g[:, :FH]
    m2[1] = g[:, FH:]


_mid_call = pl.pallas_call(
    _mid_body,
    grid=(NPAD // BM,),
    in_specs=[
        pl.BlockSpec((NC, 2, BM, FH), lambda i: (0, 0, i, 0)),
        pl.BlockSpec((BM, 1), lambda i: (i, 0)),
        pl.BlockSpec((BM, 1), lambda i: (i, 0)),
        pl.BlockSpec((F_IN, F_HID), lambda i: (0, 0)),
        pl.BlockSpec((1, F_HID), lambda i: (0, 0)),
        pl.BlockSpec((F_HID, F_OUT), lambda i: (0, 0)),
    ],
    out_specs=pl.BlockSpec((2, BM, FH), lambda i: (0, i, 0)),
    out_shape=jax.ShapeDtypeStruct((2, NPAD, FH), jnp.float32),
)


def _fin_body(q, b, b2, out):
    z = jnp.concatenate([q[0, 0, :N_NODES] + q[1, 0, :N_NODES],
                         q[0, 1, :N_NODES] + q[1, 1, :N_NODES]], axis=1)
    z = z * b[:N_NODES] + b2[...]
    out[...] = jax.nn.sigmoid(z)


_fin_call = pl.pallas_call(
    _fin_body,
    out_shape=jax.ShapeDtypeStruct((N_NODES, F_OUT), jnp.float32),
)


def kernel(x, edge_index, W1, b1, W2, b2):
    src = edge_index[0].astype(jnp.int32)
    dst = edge_index[1].astype(jnp.int32)
    pad = jnp.full((EPAD - N_EDGES,), PAD_NODE, jnp.int32)
    src3 = jnp.concatenate([src, pad]).reshape(NW, CHUNKS, CHUNK)
    dst3 = jnp.concatenate([dst, pad]).reshape(NW, CHUNKS, CHUNK)
    zrow = jnp.zeros((NPAD, FH), jnp.float32)
    zvec = jnp.zeros((NPAD,), jnp.float32)

    degp = _deg_kernel(src3, dst3, zvec)
    m1, a2, b2d = _prep_call(degp.reshape(2 * NC, NPAD), x)
    p = _agg_kernel(m1, src3, dst3, zrow)
    m2 = _mid_call(p, a2, b2d, W1, b1.reshape(1, F_HID), W2)
    q = _agg_kernel(m2, src3, dst3, zrow)
    return _fin_call(q, b2d, b2.reshape(1, F_OUT))


# speedup vs baseline: 1.0208x; 1.0093x over previous
import functools

import jax
import jax.numpy as jnp
from jax import lax
from jax.experimental import pallas as pl
from jax.experimental.pallas import tpu as pltpu
from jax.experimental.pallas import tpu_sc as plsc

N_NODES = 10000
N_EDGES = 320000
F_IN = 128
F_HID = 256
F_OUT = 128

NPAD = 10240
PAD_NODE = N_NODES
NC = 2
NS = 16
NW = NC * NS
CHUNK = 128
CHUNKS = 80
EPAD = NW * CHUNKS * CHUNK
RPT = NPAD // NS

_mesh = plsc.VectorSubcoreMesh(core_axis_name="c", subcore_axis_name="s")


@functools.partial(
    pl.kernel,
    out_type=jax.ShapeDtypeStruct((NC, 2, NPAD), jnp.float32),
    mesh=_mesh,
    scratch_types=[
        pltpu.VMEM((CHUNKS, CHUNK), jnp.int32),
        pltpu.VMEM((CHUNKS, CHUNK), jnp.int32),
        pltpu.VMEM((CHUNK,), jnp.float32),
        pltpu.VMEM_SHARED((NPAD,), jnp.float32),
        pltpu.VMEM_SHARED((NPAD,), jnp.float32),
        pltpu.SemaphoreType.DMA,
        pltpu.SemaphoreType.DMA,
    ],
)
def _deg_kernel(src_hbm, dst_hbm, zeros_hbm, out_hbm,
                src_v, dst_v, ones_v, acc_out, acc_in, dsem0, dsem1):
    cid = lax.axis_index("c")
    sid = lax.axis_index("s")
    wid = sid * NC + cid
    pltpu.sync_copy(src_hbm.at[wid], src_v)
    pltpu.sync_copy(dst_hbm.at[wid], dst_v)
    for i in range(CHUNK // 16):
        ones_v[pl.ds(16 * i, 16)] = jnp.ones((16,), jnp.float32)
    pltpu.sync_copy(zeros_hbm.at[pl.ds(sid * RPT, RPT)],
                    acc_out.at[pl.ds(sid * RPT, RPT)])
    pltpu.sync_copy(zeros_hbm.at[pl.ds(sid * RPT, RPT)],
                    acc_in.at[pl.ds(sid * RPT, RPT)])
    plsc.subcore_barrier()

    @pl.loop(0, CHUNKS)
    def _(c):
        pltpu.async_copy(ones_v, acc_out.at[src_v.at[c]], dsem0, add=True)
        pltpu.async_copy(ones_v, acc_in.at[dst_v.at[c]], dsem1, add=True)

    @pl.loop(0, CHUNKS)
    def _(c):
        pltpu.make_async_copy(ones_v, acc_out.at[src_v.at[c]], dsem0).wait()
        pltpu.make_async_copy(ones_v, acc_in.at[dst_v.at[c]], dsem1).wait()

    plsc.subcore_barrier()
    pltpu.sync_copy(acc_out.at[pl.ds(sid * RPT, RPT)],
                    out_hbm.at[cid, 0, pl.ds(sid * RPT, RPT)])
    pltpu.sync_copy(acc_in.at[pl.ds(sid * RPT, RPT)],
                    out_hbm.at[cid, 1, pl.ds(sid * RPT, RPT)])


FH = 64


def _agg_phase_body(h, cid, sid, table_hbm, zeros_hbm, out_hbm,
                    src_v, dst_v, rows_v, acc, tab_s, sems):
    pltpu.sync_copy(zeros_hbm.at[pl.ds(sid * RPT, RPT)],
                    acc.at[pl.ds(sid * RPT, RPT)])
    pltpu.sync_copy(table_hbm.at[h, pl.ds(sid * RPT, RPT)],
                    tab_s.at[pl.ds(sid * RPT, RPT)])
    plsc.subcore_barrier()

    pltpu.async_copy(tab_s.at[src_v.at[0]], rows_v.at[0], sems[0])

    @pl.loop(0, CHUNKS, step=2)
    def _(c):
        for b in range(2):
            cc = c + b
            nxt = cc + 1

            @pl.when(nxt < CHUNKS)
            def _():
                pltpu.async_copy(tab_s.at[src_v.at[nxt]],
                                 rows_v.at[1 - b], sems[1 - b])

            pltpu.make_async_copy(tab_s.at[src_v.at[cc]],
                                  rows_v.at[b], sems[b]).wait()
            pltpu.sync_copy(rows_v.at[b], acc.at[dst_v.at[cc]], add=True)

    plsc.subcore_barrier()
    pltpu.sync_copy(acc.at[pl.ds(sid * RPT, RPT)],
                    out_hbm.at[cid, h, pl.ds(sid * RPT, RPT)])


_AGG_SCRATCH = [
    pltpu.VMEM((CHUNKS, CHUNK), jnp.int32),
    pltpu.VMEM((CHUNKS, CHUNK), jnp.int32),
    pltpu.VMEM((2, CHUNK, FH), jnp.float32),
    pltpu.VMEM_SHARED((NPAD, FH), jnp.float32),
    pltpu.VMEM_SHARED((NPAD, FH), jnp.float32),
    pltpu.SemaphoreType.DMA,
    pltpu.SemaphoreType.DMA,
]


@functools.partial(
    pl.kernel,
    out_type=jax.ShapeDtypeStruct((NC, 2, NPAD, FH), jnp.float32),
    mesh=_mesh,
    scratch_types=list(_AGG_SCRATCH),
    compiler_params=pltpu.CompilerParams(use_tc_tiling_on_sc=False),
)
def _agg_kernel(table_hbm, src_hbm, dst_hbm, zeros_hbm, out_hbm,
                src_v, dst_v, rows_v, acc, tab_s, sem0, sem1):
    cid = lax.axis_index("c")
    sid = lax.axis_index("s")
    wid = sid * NC + cid
    pltpu.sync_copy(src_hbm.at[wid], src_v)
    pltpu.sync_copy(dst_hbm.at[wid], dst_v)
    for h in range(2):
        _agg_phase_body(h, cid, sid, table_hbm, zeros_hbm, out_hbm,
                        src_v, dst_v, rows_v, acc, tab_s, (sem0, sem1))


def _prep_body(dp_ref, x_ref, m1_ref, a_ref, b_ref):
    dp = dp_ref[...]
    out_deg = jnp.transpose(dp[0:1] + dp[2:3])
    in_deg = jnp.transpose(dp[1:2] + dp[3:4])
    a = lax.rsqrt(jnp.maximum(out_deg, 1.0))
    b = lax.rsqrt(jnp.maximum(in_deg, 1.0))
    a_ref[...] = a
    b_ref[...] = b
    m1 = x_ref[...] * a[:N_NODES]
    m1_ref[0, :N_NODES] = m1[:, :FH]
    m1_ref[1, :N_NODES] = m1[:, FH:]
    m1_ref[0, N_NODES:] = jnp.zeros((NPAD - N_NODES, FH), jnp.float32)
    m1_ref[1, N_NODES:] = jnp.zeros((NPAD - N_NODES, FH), jnp.float32)


_prep_call = pl.pallas_call(
    _prep_body,
    out_shape=[
        jax.ShapeDtypeStruct((2, NPAD, FH), jnp.float32),
        jax.ShapeDtypeStruct((NPAD, 1), jnp.float32),
        jax.ShapeDtypeStruct((NPAD, 1), jnp.float32),
    ],
)

BM = 1024


def _mid_body(p, a, b, W1, b1, W2, m2):
    agg = jnp.concatenate([p[0, 0] + p[1, 0], p[0, 1] + p[1, 1]], axis=1)
    agg = agg * b[...]
    h = jnp.dot(agg, W1[...], preferred_element_type=jnp.float32) + b1[...]
    h = jnp.where(h >= 0, h, 0.01 * h)
    g = jnp.dot(h, W2[...], preferred_element_type=jnp.float32)
    g = g * a[...]
    m2[0] = g[:, :FH]
    m2[1] = g[:, FH:]


_mid_call = pl.pallas_call(
    _mid_body,
    grid=(NPAD // BM,),
    in_specs=[
        pl.BlockSpec((NC, 2, BM, FH), lambda i: (0, 0, i, 0)),
        pl.BlockSpec((BM, 1), lambda i: (i, 0)),
        pl.BlockSpec((BM, 1), lambda i: (i, 0)),
        pl.BlockSpec((F_IN, F_HID), lambda i: (0, 0)),
        pl.BlockSpec((1, F_HID), lambda i: (0, 0)),
        pl.BlockSpec((F_HID, F_OUT), lambda i: (0, 0)),
    ],
    out_specs=pl.BlockSpec((2, BM, FH), lambda i: (0, i, 0)),
    out_shape=jax.ShapeDtypeStruct((2, NPAD, FH), jnp.float32),
)


def _fin_body(q, b, b2, out):
    z = jnp.concatenate([q[0, 0, :N_NODES] + q[1, 0, :N_NODES],
                         q[0, 1, :N_NODES] + q[1, 1, :N_NODES]], axis=1)
    z = z * b[:N_NODES] + b2[...]
    out[...] = jax.nn.sigmoid(z)


_fin_call = pl.pallas_call(
    _fin_body,
    out_shape=jax.ShapeDtypeStruct((N_NODES, F_OUT), jnp.float32),
)


def kernel(x, edge_index, W1, b1, W2, b2):
    src = edge_index[0].astype(jnp.int32)
    dst = edge_index[1].astype(jnp.int32)
    pad = jnp.full((EPAD - N_EDGES,), PAD_NODE, jnp.int32)
    src3 = jnp.concatenate([src, pad]).reshape(NW, CHUNKS, CHUNK)
    dst3 = jnp.concatenate([dst, pad]).reshape(NW, CHUNKS, CHUNK)
    zrow = jnp.zeros((NPAD, FH), jnp.float32)
    zvec = jnp.zeros((NPAD,), jnp.float32)

    degp = _deg_kernel(src3, dst3, zvec)
    m1, a2, b2d = _prep_call(degp.reshape(2 * NC, NPAD), x)
    p = _agg_kernel(m1, src3, dst3, zrow)
    m2 = _mid_call(p, a2, b2d, W1, b1.reshape(1, F_HID), W2)
    q = _agg_kernel(m2, src3, dst3, zrow)
    return _fin_call(q, b2d, b2.reshape(1, F_OUT))
